# initial kernel scaffold (unmeasured)
import jax
import jax.numpy as jnp
from jax import lax
from jax.experimental import pallas as pl
from jax.experimental.pallas import tpu as pltpu


def kernel(
    x,
):
    def body(*refs):
        pass

    out_shape = jax.ShapeDtypeStruct(..., jnp.float32)
    return pl.pallas_call(body, out_shape=out_shape)(...)


# baseline (device time: 339336 ns/iter reference)
import jax
import jax.numpy as jnp
from jax import lax
from jax.experimental import pallas as pl
from jax.experimental.pallas import tpu as pltpu

N_DEV = 4


def kernel(x):
    x = x.astype(jnp.bfloat16)
    m_per, n = x.shape
    half = m_per // 2

    def body(x_ref, out_ref, local_sem, fwd_send, fwd_recv, bwd_send, bwd_recv):
        my = lax.axis_index("i")
        left = lax.rem(my + N_DEV - 1, N_DEV)
        right = lax.rem(my + 1, N_DEV)

        barrier = pltpu.get_barrier_semaphore()
        for nbr in (left, right):
            pl.semaphore_signal(
                barrier, inc=1,
                device_id=(nbr,), device_id_type=pl.DeviceIdType.MESH,
            )
        pl.semaphore_wait(barrier, 2)

        cp = pltpu.make_async_copy(
            x_ref, out_ref.at[pl.ds(my * m_per, m_per)], local_sem
        )
        cp.start()
        cp.wait()

        for h in range(N_DEV - 1):
            fo = lax.rem(my - h + N_DEV, N_DEV)
            bo = lax.rem(my + h, N_DEV)
            fwd = pltpu.make_async_remote_copy(
                src_ref=out_ref.at[pl.ds(fo * m_per, half)],
                dst_ref=out_ref.at[pl.ds(fo * m_per, half)],
                send_sem=fwd_send.at[h],
                recv_sem=fwd_recv.at[h],
                device_id=(right,),
                device_id_type=pl.DeviceIdType.MESH,
            )
            bwd = pltpu.make_async_remote_copy(
                src_ref=out_ref.at[pl.ds(bo * m_per + half, half)],
                dst_ref=out_ref.at[pl.ds(bo * m_per + half, half)],
                send_sem=bwd_send.at[h],
                recv_sem=bwd_recv.at[h],
                device_id=(left,),
                device_id_type=pl.DeviceIdType.MESH,
            )
            fwd.start()
            bwd.start()
            fwd.wait()
            bwd.wait()

    return pl.pallas_call(
        body,
        out_shape=jax.ShapeDtypeStruct((N_DEV * m_per, n), jnp.bfloat16),
        in_specs=[pl.BlockSpec(memory_space=pl.ANY)],
        out_specs=pl.BlockSpec(memory_space=pl.ANY),
        scratch_shapes=[
            pltpu.SemaphoreType.DMA,
            pltpu.SemaphoreType.DMA((N_DEV - 1,)),
            pltpu.SemaphoreType.DMA((N_DEV - 1,)),
            pltpu.SemaphoreType.DMA((N_DEV - 1,)),
            pltpu.SemaphoreType.DMA((N_DEV - 1,)),
        ],
        compiler_params=pltpu.CompilerParams(collective_id=0),
    )(x)


# device time: 333217 ns/iter; 1.0184x vs baseline; 1.0184x over previous
import jax
import jax.numpy as jnp
from jax import lax
from jax.experimental import pallas as pl
from jax.experimental.pallas import tpu as pltpu

N_DEV = 4


def kernel(x):
    x = x.astype(jnp.bfloat16)
    m_per, n = x.shape
    half = m_per // 2

    def body(x_ref, out_ref, local_sem, fwd_send, fwd_recv, bwd_send, bwd_recv):
        my = lax.axis_index("i")
        left = lax.rem(my + N_DEV - 1, N_DEV)
        right = lax.rem(my + 1, N_DEV)

        barrier = pltpu.get_barrier_semaphore()
        for nbr in (left, right):
            pl.semaphore_signal(
                barrier, inc=1,
                device_id=(nbr,), device_id_type=pl.DeviceIdType.MESH,
            )
        pl.semaphore_wait(barrier, 2)

        cp = pltpu.make_async_copy(
            x_ref, out_ref.at[pl.ds(my * m_per, m_per)], local_sem
        )
        cp.start()

        fwds, bwds = [], []
        for h in range(N_DEV - 1):
            fo = lax.rem(my - h + N_DEV, N_DEV)
            bo = lax.rem(my + h, N_DEV)
            fwd = pltpu.make_async_remote_copy(
                src_ref=x_ref.at[pl.ds(0, half)] if h == 0
                else out_ref.at[pl.ds(fo * m_per, half)],
                dst_ref=out_ref.at[pl.ds(fo * m_per, half)],
                send_sem=fwd_send.at[h],
                recv_sem=fwd_recv.at[h],
                device_id=(right,),
                device_id_type=pl.DeviceIdType.MESH,
            )
            bwd = pltpu.make_async_remote_copy(
                src_ref=x_ref.at[pl.ds(half, half)] if h == 0
                else out_ref.at[pl.ds(bo * m_per + half, half)],
                dst_ref=out_ref.at[pl.ds(bo * m_per + half, half)],
                send_sem=bwd_send.at[h],
                recv_sem=bwd_recv.at[h],
                device_id=(left,),
                device_id_type=pl.DeviceIdType.MESH,
            )
            if h > 0:
                fwds[h - 1].wait_recv()
                bwds[h - 1].wait_recv()
            fwd.start()
            bwd.start()
            fwds.append(fwd)
            bwds.append(bwd)

        fwds[-1].wait_recv()
        bwds[-1].wait_recv()
        for h in range(N_DEV - 1):
            fwds[h].wait_send()
            bwds[h].wait_send()
        cp.wait()

    return pl.pallas_call(
        body,
        out_shape=jax.ShapeDtypeStruct((N_DEV * m_per, n), jnp.bfloat16),
        in_specs=[pl.BlockSpec(memory_space=pl.ANY)],
        out_specs=pl.BlockSpec(memory_space=pl.ANY),
        scratch_shapes=[
            pltpu.SemaphoreType.DMA,
            pltpu.SemaphoreType.DMA((N_DEV - 1,)),
            pltpu.SemaphoreType.DMA((N_DEV - 1,)),
            pltpu.SemaphoreType.DMA((N_DEV - 1,)),
            pltpu.SemaphoreType.DMA((N_DEV - 1,)),
        ],
        compiler_params=pltpu.CompilerParams(collective_id=0),
    )(x)


# device time: 328271 ns/iter; 1.0337x vs baseline; 1.0151x over previous
import jax
import jax.numpy as jnp
from jax import lax
from jax.experimental import pallas as pl
from jax.experimental.pallas import tpu as pltpu

N_DEV = 4


def kernel(x):
    x = x.astype(jnp.bfloat16)
    m_per, n = x.shape
    half = m_per // 2

    def body(x_ref, out_ref, local_sem, send_r, recv_l, send_l, recv_r):
        my = lax.axis_index("i")
        left = lax.rem(my + N_DEV - 1, N_DEV)
        right = lax.rem(my + 1, N_DEV)

        barrier = pltpu.get_barrier_semaphore()
        for nbr in (left, right):
            pl.semaphore_signal(
                barrier, inc=1,
                device_id=(nbr,), device_id_type=pl.DeviceIdType.MESH,
            )
        pl.semaphore_wait(barrier, 2)

        cp = pltpu.make_async_copy(
            x_ref, out_ref.at[pl.ds(my * m_per, m_per)], local_sem
        )
        cp.start()

        def rdma(src, dst, ssem, rsem, dev):
            return pltpu.make_async_remote_copy(
                src_ref=src, dst_ref=dst, send_sem=ssem, recv_sem=rsem,
                device_id=(dev,), device_id_type=pl.DeviceIdType.MESH,
            )

        r0 = rdma(x_ref.at[pl.ds(0, half)],
                  out_ref.at[pl.ds(my * m_per, half)],
                  send_r.at[0], recv_l.at[0], right)
        r1 = rdma(x_ref.at[pl.ds(half, half)],
                  out_ref.at[pl.ds(my * m_per + half, half)],
                  send_r.at[1], recv_l.at[1], right)
        r2 = rdma(out_ref.at[pl.ds(left * m_per, half)],
                  out_ref.at[pl.ds(left * m_per, half)],
                  send_r.at[2], recv_l.at[2], right)
        l0 = rdma(x_ref.at[pl.ds(half, half)],
                  out_ref.at[pl.ds(my * m_per + half, half)],
                  send_l.at[0], recv_r.at[0], left)
        l1 = rdma(x_ref.at[pl.ds(0, half)],
                  out_ref.at[pl.ds(my * m_per, half)],
                  send_l.at[1], recv_r.at[1], left)
        l2 = rdma(out_ref.at[pl.ds(right * m_per + half, half)],
                  out_ref.at[pl.ds(right * m_per + half, half)],
                  send_l.at[2], recv_r.at[2], left)

        r0.start()
        r1.start()
        l0.start()
        l1.start()

        r0.wait_recv()
        r2.start()
        l0.wait_recv()
        l2.start()

        r1.wait_recv()
        l1.wait_recv()
        r2.wait_recv()
        l2.wait_recv()
        for d in (r0, r1, r2, l0, l1, l2):
            d.wait_send()
        cp.wait()

    return pl.pallas_call(
        body,
        out_shape=jax.ShapeDtypeStruct((N_DEV * m_per, n), jnp.bfloat16),
        in_specs=[pl.BlockSpec(memory_space=pl.ANY)],
        out_specs=pl.BlockSpec(memory_space=pl.ANY),
        scratch_shapes=[
            pltpu.SemaphoreType.DMA,
            pltpu.SemaphoreType.DMA((3,)),
            pltpu.SemaphoreType.DMA((3,)),
            pltpu.SemaphoreType.DMA((3,)),
            pltpu.SemaphoreType.DMA((3,)),
        ],
        compiler_params=pltpu.CompilerParams(collective_id=0),
    )(x)


# device time: 322521 ns/iter; 1.0521x vs baseline; 1.0178x over previous
import jax
import jax.numpy as jnp
from jax import lax
from jax.experimental import pallas as pl
from jax.experimental.pallas import tpu as pltpu

N_DEV = 4
N_CHUNK = 4


def kernel(x):
    m_per, n = x.shape
    half = m_per // 2
    mc = m_per // N_CHUNK

    def body(x_ref, out_ref, xf32, xbf, chunk_sems, local_sem,
             send_r, recv_l, send_l, recv_r):
        my = lax.axis_index("i")
        left = lax.rem(my + N_DEV - 1, N_DEV)
        right = lax.rem(my + 1, N_DEV)

        barrier = pltpu.get_barrier_semaphore()
        for nbr in (left, right):
            pl.semaphore_signal(
                barrier, inc=1,
                device_id=(nbr,), device_id_type=pl.DeviceIdType.MESH,
            )
        pl.semaphore_wait(barrier, 2)

        loads = []
        for c in (0, 2, 1, 3):
            ld = pltpu.make_async_copy(
                x_ref.at[pl.ds(c * mc, mc)], xf32.at[pl.ds(c * mc, mc)],
                chunk_sems.at[c],
            )
            ld.start()
            loads.append((c, ld))

        def rdma(src, dst, ssem, rsem, dev):
            return pltpu.make_async_remote_copy(
                src_ref=src, dst_ref=dst, send_sem=ssem, recv_sem=rsem,
                device_id=(dev,), device_id_type=pl.DeviceIdType.MESH,
            )

        r_slot = {0: 0, 1: 1, 2: 2, 3: 3}
        l_slot = {2: 0, 3: 1, 0: 2, 1: 3}

        rs, ls = {}, {}
        for c, ld in loads:
            ld.wait()
            xbf[pl.ds(c * mc, mc)] = xf32[pl.ds(c * mc, mc)].astype(jnp.bfloat16)
            src = xbf.at[pl.ds(c * mc, mc)]
            dst = out_ref.at[pl.ds(my * m_per + c * mc, mc)]
            rk, lk = r_slot[c], l_slot[c]
            rs[rk] = rdma(src, dst, send_r.at[rk], recv_l.at[rk], right)
            ls[lk] = rdma(src, dst, send_l.at[lk], recv_r.at[lk], left)
            rs[rk].start()
            ls[lk].start()

        cp = pltpu.make_async_copy(
            xbf, out_ref.at[pl.ds(my * m_per, m_per)], local_sem
        )
        cp.start()

        rs[0].wait_recv()
        rs[1].wait_recv()
        r4 = rdma(out_ref.at[pl.ds(left * m_per, half)],
                  out_ref.at[pl.ds(left * m_per, half)],
                  send_r.at[4], recv_l.at[4], right)
        r4.start()
        ls[0].wait_recv()
        ls[1].wait_recv()
        l4 = rdma(out_ref.at[pl.ds(right * m_per + half, half)],
                  out_ref.at[pl.ds(right * m_per + half, half)],
                  send_l.at[4], recv_r.at[4], left)
        l4.start()

        rs[2].wait_recv()
        rs[3].wait_recv()
        ls[2].wait_recv()
        ls[3].wait_recv()
        r4.wait_recv()
        l4.wait_recv()
        for d in (*rs.values(), *ls.values(), r4, l4):
            d.wait_send()
        cp.wait()

    return pl.pallas_call(
        body,
        out_shape=jax.ShapeDtypeStruct((N_DEV * m_per, n), jnp.bfloat16),
        in_specs=[pl.BlockSpec(memory_space=pl.ANY)],
        out_specs=pl.BlockSpec(memory_space=pl.ANY),
        scratch_shapes=[
            pltpu.VMEM((m_per, n), jnp.float32),
            pltpu.VMEM((m_per, n), jnp.bfloat16),
            pltpu.SemaphoreType.DMA((N_CHUNK,)),
            pltpu.SemaphoreType.DMA,
            pltpu.SemaphoreType.DMA((5,)),
            pltpu.SemaphoreType.DMA((5,)),
            pltpu.SemaphoreType.DMA((5,)),
            pltpu.SemaphoreType.DMA((5,)),
        ],
        compiler_params=pltpu.CompilerParams(
            collective_id=0,
            vmem_limit_bytes=56 * 1024 * 1024,
        ),
    )(x)
